# baseline (speedup 1.0000x reference)
import functools

import jax
import jax.numpy as jnp
from jax import lax
from jax.experimental import pallas as pl
from jax.experimental.pallas import tpu as pltpu
from jax.experimental.pallas import tpu_sc as plsc

B = 16384
D = 64
NE = 1000000
NR = 1000
NC = 2
NS = 16
NW = NC * NS
BPW = B // NW
L = 16

BW = 512
BSH = 9
NFB = NE // BW
TAIL0 = NFB * BW
NBLK = NFB + 1
CAP = 64
MAXBPW = NFB // NW + 1
FBPW = NFB // NW
NXTRA = NFB - FBPW * NW
G1 = 2 * B
SPARE = G1
N1 = G1 + 128


SROWS = 64


def _flush(staging, posrow, out1_hbm, sem, iota):
    cp = pltpu.async_copy(staging, out1_hbm.at[posrow.at[0]], sem)
    cp.wait()
    for v in range(SROWS // L):
        posrow[0, pl.ds(v * L, L)] = jnp.full((L,), SPARE + v * L, jnp.int32) + iota


def _gather_body(heads_hbm, tails_hbm, et_hbm, out1_hbm,
                 src_v, ovf, buckets, bcnt, blockbuf, staging, posrow,
                 sem_a, sem_b, sem_s):
    wid = lax.axis_index("s") * NC + lax.axis_index("c")
    start = wid * FBPW + jnp.minimum(wid, NXTRA)
    cnt = FBPW + (wid < NXTRA).astype(jnp.int32)
    lo = start * BW
    hi = jnp.where(wid == NW - 1, NE, (start + cnt) * BW)
    iota = lax.iota(jnp.int32, L)

    for v in range(SROWS // L):
        posrow[0, pl.ds(v * L, L)] = jnp.full((L,), SPARE + v * L, jnp.int32) + iota
    for v in range(128 // L):
        bcnt[pl.ds(v * L, L)] = jnp.zeros((L,), jnp.int32)

    def filter_list(src_hbm, posbase, novf):
        pltpu.sync_copy(src_hbm, src_v)

        def fv(v, novf):
            idx = src_v[pl.ds(v * L, L)]
            inr = (idx >= lo) & (idx < hi)
            lidx = idx - lo
            pos = posbase + v * L + iota
            packed = (lidx << 16) | pos
            bkt = lidx >> BSH
            base = plsc.load_gather(bcnt, [jnp.where(inr, bkt, 0)])
            rank, _ = plsc.scan_count(bkt, inr)
            slot = base + rank - 1
            keep = inr & (slot < CAP)
            spill = inr & (slot >= CAP)
            plsc.store_scatter(buckets, [bkt * CAP + slot], packed, mask=keep)
            plsc.addupdate_scatter(bcnt, [bkt], inr.astype(jnp.int32),
                                   mask=inr)
            pref = plsc.cumsum(spill.astype(jnp.int32))
            plsc.store_scatter(ovf, [novf + pref - 1], packed, mask=spill)
            return novf + pref[15]

        return lax.fori_loop(0, B // L, fv, novf)

    novf = filter_list(heads_hbm, 0, jnp.int32(0))
    novf = filter_list(tails_hbm, B, novf)

    def extract_event(buf, hits16, mask, n):
        @pl.when(n > SROWS - L)
        def _():
            _flush(staging, posrow, out1_hbm, sem_s, iota)

        n = jnp.where(n > SROWS - L, 0, n)
        bidx = (hits16 >> 16) & (BW - 1)
        pos = hits16 & 0xFFFF
        pref = plsc.cumsum(mask.astype(jnp.int32))
        slot = n + pref - 1
        plsc.store_scatter(posrow, [jnp.zeros((L,), jnp.int32), slot], pos,
                           mask=mask)
        sbase = slot * 128
        dv = jnp.zeros((L,), jnp.int32)
        for d in range(D):
            vals = plsc.load_gather(buf, [dv, bidx])
            plsc.store_scatter(staging, [slot, dv], vals, mask=mask)
            dv = dv + 1
        return n + pref[15]

    def process_block(i, buf, n):
        cnt_i = plsc.load_gather(bcnt, [jnp.full((L,), i, jnp.int32)])[0]
        cnt_eff = jnp.minimum(cnt_i, CAP)

        def bg(g, n):
            hits16 = buckets[pl.ds(i * CAP + g * L, L)]
            mask = (g * L + iota) < cnt_eff
            return extract_event(buf, hits16, mask, n)

        n = lax.fori_loop(0, (cnt_eff + L - 1) // L, bg, n)

        def ov(v, n):
            hits16 = ovf[pl.ds(v * L, L)]
            mask = ((hits16 >> 16) >> 8) == i
            mask = mask & ((v * L + iota) < novf)
            return extract_event(buf, hits16, mask, n)

        n = lax.cond(cnt_i > CAP,
                     lambda n: lax.fori_loop(0, (novf + L - 1) // L, ov, n),
                     lambda n: n, n)
        return n

    def issue(i, buf, sem):
        blk = pl.multiple_of((start + i) * BW, 128)
        return pltpu.async_copy(et_hbm.at[:, pl.ds(blk, BW)], buf, sem)

    def wait_like(buf, sem):
        pltpu.make_async_copy(et_hbm.at[:, pl.ds(0, BW)], buf, sem).wait()

    issue(0, blockbuf.at[0], sem_a)

    def step(i2, n):
        i = i2 * 2

        def one(i, buf, sem, obuf, osem, n):
            @pl.when(i + 1 < cnt)
            def _():
                issue(i + 1, obuf, osem)
            wait_like(buf, sem)
            return process_block(i, buf, n)

        n = one(i, blockbuf.at[0], sem_a, blockbuf.at[1], sem_b, n)
        n = lax.cond(i + 1 < cnt,
                     lambda n: one(i + 1, blockbuf.at[1], sem_b,
                                   blockbuf.at[0], sem_a, n),
                     lambda n: n, n)
        return n

    n = lax.fori_loop(0, (cnt + 1) // 2, step, jnp.int32(0))

    @pl.when(wid == NW - 1)
    def _():
        toff = pl.multiple_of((hi >> 7) << 7, 128)
        pltpu.sync_copy(et_hbm.at[:, pl.ds(toff, 128)],
                        blockbuf.at[0, :, pl.ds(0, 128)])

    n = lax.cond(
        wid == NW - 1,
        lambda n: process_block(cnt, blockbuf.at[0], n),
        lambda n: n, n)

    @pl.when(n > 0)
    def _():
        _flush(staging, posrow, out1_hbm, sem_s, iota)


def _norm_body(rels_hbm, remb_hbm, g_hbm, out_hbm,
               idx_r, relbuf, hchunk, tchunk, sums, sem_h, sem_t):
    wid = lax.axis_index("s") * NC + lax.axis_index("c")
    base = wid * BPW
    iota = lax.iota(jnp.int32, L)

    pltpu.sync_copy(rels_hbm.at[pl.ds(base, BPW)], idx_r)
    pltpu.sync_copy(remb_hbm, relbuf)

    CH = 64

    def issue(c, k):
        pltpu.async_copy(g_hbm.at[pl.ds(base + c * CH, CH), :],
                         hchunk.at[k], sem_h)
        pltpu.async_copy(g_hbm.at[pl.ds(B + base + c * CH, CH), :],
                         tchunk.at[k], sem_t)

    def wait_chunk(k):
        pltpu.make_async_copy(g_hbm.at[pl.ds(0, CH), :], hchunk.at[k],
                              sem_h).wait()
        pltpu.make_async_copy(g_hbm.at[pl.ds(0, CH), :], tchunk.at[k],
                              sem_t).wait()

    issue(0, 0)

    def chunk(c, k, carry):
        @pl.when(c + 1 < BPW // CH)
        def _():
            issue(c + 1, 1 - k)
        wait_chunk(k)

        def grp(g, carry):
            row = g * L + iota
            rv = idx_r[pl.ds(c * CH + g * L, L)]
            accs = [jnp.zeros((L,), jnp.float32) for _ in range(4)]
            dv = jnp.zeros((L,), jnp.int32)
            for d in range(D):
                h = plsc.load_gather(hchunk.at[k], [row, dv])
                t = plsc.load_gather(tchunk.at[k], [row, dv])
                r = plsc.load_gather(relbuf, [dv, rv])
                dd = (h + r) - t
                accs[d % 4] = accs[d % 4] + dd * dd
                dv = dv + 1
            x = (accs[0] + accs[1]) + (accs[2] + accs[3])
            xi = plsc.bitcast(x, jnp.int32)
            y = plsc.bitcast(jnp.int32(0x5F3759DF) - (xi >> 1), jnp.float32)
            for _ in range(3):
                y = y * (1.5 - 0.5 * x * y * y)
            sums[pl.ds(c * CH + g * L, L)] = x * y
            return carry

        return lax.fori_loop(0, CH // L, grp, carry)

    def pair(c2, carry):
        carry = chunk(c2 * 2, 0, carry)
        return chunk(c2 * 2 + 1, 1, carry)

    lax.fori_loop(0, BPW // CH // 2, pair, 0)
    pltpu.sync_copy(sums, out_hbm.at[pl.ds(base, BPW)])


@jax.jit
def _transe_sc(heads, relations, tails, entity_t, relation_t):
    mesh = plsc.VectorSubcoreMesh(core_axis_name="c", subcore_axis_name="s")
    params = pltpu.CompilerParams(
        needs_layout_passes=False, use_tc_tiling_on_sc=True)

    gather = functools.partial(
        pl.kernel, mesh=mesh,
        out_type=jax.ShapeDtypeStruct((N1, 128), jnp.float32),
        compiler_params=params,
        scratch_types=[
            pltpu.VMEM((B,), jnp.int32),
            pltpu.VMEM((G1,), jnp.int32),
            pltpu.VMEM((MAXBPW * CAP,), jnp.int32),
            pltpu.VMEM((128,), jnp.int32),
            pltpu.VMEM((2, D, BW), jnp.float32),
            pltpu.VMEM((SROWS, 128), jnp.float32),
            pltpu.VMEM((1, SROWS), jnp.int32),
            pltpu.SemaphoreType.DMA,
            pltpu.SemaphoreType.DMA,
            pltpu.SemaphoreType.DMA,
        ],
    )(_gather_body)
    g = gather(heads, tails, entity_t)

    norm = functools.partial(
        pl.kernel, mesh=mesh,
        out_type=jax.ShapeDtypeStruct((B,), jnp.float32),
        compiler_params=params,
        scratch_types=[
            pltpu.VMEM((BPW,), jnp.int32),
            pltpu.VMEM((D, NR), jnp.float32),
            pltpu.VMEM((2, 64, 128), jnp.float32),
            pltpu.VMEM((2, 64, 128), jnp.float32),
            pltpu.VMEM((BPW,), jnp.float32),
            pltpu.SemaphoreType.DMA,
            pltpu.SemaphoreType.DMA,
        ],
    )(_norm_body)
    return norm(relations, relation_t, g)


def kernel(heads, relations, tails, entity_emb, relation_emb):
    heads = heads.astype(jnp.int32)
    relations = relations.astype(jnp.int32)
    tails = tails.astype(jnp.int32)
    return _transe_sc(heads, relations, tails, entity_emb.T, relation_emb.T)


# speedup vs baseline: 1.0595x; 1.0595x over previous
import functools

import jax
import jax.numpy as jnp
from jax import lax
from jax.experimental import pallas as pl
from jax.experimental.pallas import tpu as pltpu
from jax.experimental.pallas import tpu_sc as plsc

B = 16384
D = 64
NE = 1000000
NR = 1000
NC = 2
NS = 16
NW = NC * NS
BPW = B // NW
L = 16

BW = 256
NFB = NE // BW
TAIL0 = NFB * BW
NBLK = NFB + 1
CAP = 32
MAXBPW = NFB // NW + 1
G1 = 2 * B
SPARE = G1
N1 = G1 + 128


SROWS = 128


def _flush(staging, posrow, out1_hbm, sem, iota):
    cp = pltpu.async_copy(staging, out1_hbm.at[posrow.at[0]], sem)
    cp.wait()
    for v in range(SROWS // L):
        posrow[0, pl.ds(v * L, L)] = jnp.full((L,), SPARE + v * L, jnp.int32) + iota


def _gather_body(heads_hbm, tails_hbm, et_hbm, out1_hbm,
                 src_v, ovf, buckets, bcnt, blockbuf, staging, posrow,
                 sem_a, sem_b, sem_s):
    wid = lax.axis_index("s") * NC + lax.axis_index("c")
    start = wid * 122 + jnp.minimum(wid, 2)
    cnt = 122 + (wid < 2).astype(jnp.int32)
    lo = start * BW
    hi = jnp.where(wid == NW - 1, NE, (start + cnt) * BW)
    iota = lax.iota(jnp.int32, L)

    for v in range(SROWS // L):
        posrow[0, pl.ds(v * L, L)] = jnp.full((L,), SPARE + v * L, jnp.int32) + iota
    for v in range(128 // L):
        bcnt[pl.ds(v * L, L)] = jnp.zeros((L,), jnp.int32)

    def filter_list(src_hbm, posbase, novf):
        pltpu.sync_copy(src_hbm, src_v)

        def fv(v, novf):
            idx = src_v[pl.ds(v * L, L)]
            inr = (idx >= lo) & (idx < hi)
            lidx = idx - lo
            pos = posbase + v * L + iota
            packed = (lidx << 16) | pos
            bkt = lidx >> 8
            base = plsc.load_gather(bcnt, [jnp.where(inr, bkt, 0)])
            rank, _ = plsc.scan_count(bkt, inr)
            slot = base + rank - 1
            keep = inr & (slot < CAP)
            spill = inr & (slot >= CAP)
            plsc.store_scatter(buckets, [bkt * CAP + slot], packed, mask=keep)
            plsc.addupdate_scatter(bcnt, [bkt], inr.astype(jnp.int32),
                                   mask=inr)
            pref = plsc.cumsum(spill.astype(jnp.int32))
            plsc.store_scatter(ovf, [novf + pref - 1], packed, mask=spill)
            return novf + pref[15]

        return lax.fori_loop(0, B // L, fv, novf)

    novf = filter_list(heads_hbm, 0, jnp.int32(0))
    novf = filter_list(tails_hbm, B, novf)

    def extract_event(buf, hits16, mask, n):
        @pl.when(n > SROWS - L)
        def _():
            _flush(staging, posrow, out1_hbm, sem_s, iota)

        n = jnp.where(n > SROWS - L, 0, n)
        bidx = (hits16 >> 16) & (BW - 1)
        pos = hits16 & 0xFFFF
        pref = plsc.cumsum(mask.astype(jnp.int32))
        slot = n + pref - 1
        plsc.store_scatter(posrow, [jnp.zeros((L,), jnp.int32), slot], pos,
                           mask=mask)
        sbase = slot * 128
        dv = jnp.zeros((L,), jnp.int32)
        for d in range(D):
            vals = plsc.load_gather(buf, [dv, bidx])
            plsc.store_scatter(staging, [slot, dv], vals, mask=mask)
            dv = dv + 1
        return n + pref[15]

    def process_block(i, buf, n):
        cnt_i = plsc.load_gather(bcnt, [jnp.full((L,), i, jnp.int32)])[0]
        cnt_eff = jnp.minimum(cnt_i, CAP)

        def bg(g, n):
            hits16 = buckets[pl.ds(i * CAP + g * L, L)]
            mask = (g * L + iota) < cnt_eff
            return extract_event(buf, hits16, mask, n)

        n = lax.fori_loop(0, (cnt_eff + L - 1) // L, bg, n)

        def ov(v, n):
            hits16 = ovf[pl.ds(v * L, L)]
            mask = ((hits16 >> 16) >> 8) == i
            mask = mask & ((v * L + iota) < novf)
            return extract_event(buf, hits16, mask, n)

        n = lax.cond(cnt_i > CAP,
                     lambda n: lax.fori_loop(0, (novf + L - 1) // L, ov, n),
                     lambda n: n, n)
        return n

    def issue(i, buf, sem):
        blk = pl.multiple_of((start + i) * BW, 128)
        return pltpu.async_copy(et_hbm.at[:, pl.ds(blk, BW)], buf, sem)

    def wait_like(buf, sem):
        pltpu.make_async_copy(et_hbm.at[:, pl.ds(0, BW)], buf, sem).wait()

    issue(0, blockbuf.at[0], sem_a)

    def step(i2, n):
        i = i2 * 2

        def one(i, buf, sem, obuf, osem, n):
            @pl.when(i + 1 < cnt)
            def _():
                issue(i + 1, obuf, osem)
            wait_like(buf, sem)
            return process_block(i, buf, n)

        n = one(i, blockbuf.at[0], sem_a, blockbuf.at[1], sem_b, n)
        n = lax.cond(i + 1 < cnt,
                     lambda n: one(i + 1, blockbuf.at[1], sem_b,
                                   blockbuf.at[0], sem_a, n),
                     lambda n: n, n)
        return n

    n = lax.fori_loop(0, (cnt + 1) // 2, step, jnp.int32(0))

    @pl.when(wid == NW - 1)
    def _():
        toff = pl.multiple_of((hi >> 7) << 7, 128)
        pltpu.sync_copy(et_hbm.at[:, pl.ds(toff, 128)],
                        blockbuf.at[0, :, pl.ds(0, 128)])

    n = lax.cond(
        wid == NW - 1,
        lambda n: process_block(cnt, blockbuf.at[0], n),
        lambda n: n, n)

    @pl.when(n > 0)
    def _():
        _flush(staging, posrow, out1_hbm, sem_s, iota)


def _norm_body(rels_hbm, remb_hbm, g_hbm, out_hbm,
               idx_r, relbuf, hchunk, tchunk, sums, sem_h, sem_t):
    wid = lax.axis_index("s") * NC + lax.axis_index("c")
    base = wid * BPW
    iota = lax.iota(jnp.int32, L)

    pltpu.sync_copy(rels_hbm.at[pl.ds(base, BPW)], idx_r)
    pltpu.sync_copy(remb_hbm, relbuf)

    CH = 64

    def issue(c, k):
        pltpu.async_copy(g_hbm.at[pl.ds(base + c * CH, CH), :],
                         hchunk.at[k], sem_h)
        pltpu.async_copy(g_hbm.at[pl.ds(B + base + c * CH, CH), :],
                         tchunk.at[k], sem_t)

    def wait_chunk(k):
        pltpu.make_async_copy(g_hbm.at[pl.ds(0, CH), :], hchunk.at[k],
                              sem_h).wait()
        pltpu.make_async_copy(g_hbm.at[pl.ds(0, CH), :], tchunk.at[k],
                              sem_t).wait()

    issue(0, 0)

    def chunk(c, k, carry):
        @pl.when(c + 1 < BPW // CH)
        def _():
            issue(c + 1, 1 - k)
        wait_chunk(k)

        def grp(g, carry):
            row = g * L + iota
            rv = idx_r[pl.ds(c * CH + g * L, L)]
            accs = [jnp.zeros((L,), jnp.float32) for _ in range(4)]
            dv = jnp.zeros((L,), jnp.int32)
            for d in range(D):
                h = plsc.load_gather(hchunk.at[k], [row, dv])
                t = plsc.load_gather(tchunk.at[k], [row, dv])
                r = plsc.load_gather(relbuf, [dv, rv])
                dd = (h + r) - t
                accs[d % 4] = accs[d % 4] + dd * dd
                dv = dv + 1
            x = (accs[0] + accs[1]) + (accs[2] + accs[3])
            xi = plsc.bitcast(x, jnp.int32)
            y = plsc.bitcast(jnp.int32(0x5F3759DF) - (xi >> 1), jnp.float32)
            for _ in range(3):
                y = y * (1.5 - 0.5 * x * y * y)
            sums[pl.ds(c * CH + g * L, L)] = x * y
            return carry

        return lax.fori_loop(0, CH // L, grp, carry)

    def pair(c2, carry):
        carry = chunk(c2 * 2, 0, carry)
        return chunk(c2 * 2 + 1, 1, carry)

    lax.fori_loop(0, BPW // CH // 2, pair, 0)
    pltpu.sync_copy(sums, out_hbm.at[pl.ds(base, BPW)])


@jax.jit
def _transe_sc(heads, relations, tails, entity_t, relation_t):
    mesh = plsc.VectorSubcoreMesh(core_axis_name="c", subcore_axis_name="s")
    params = pltpu.CompilerParams(
        needs_layout_passes=False, use_tc_tiling_on_sc=True)

    gather = functools.partial(
        pl.kernel, mesh=mesh,
        out_type=jax.ShapeDtypeStruct((N1, 128), jnp.float32),
        compiler_params=params,
        scratch_types=[
            pltpu.VMEM((B,), jnp.int32),
            pltpu.VMEM((G1,), jnp.int32),
            pltpu.VMEM((MAXBPW * CAP,), jnp.int32),
            pltpu.VMEM((128,), jnp.int32),
            pltpu.VMEM((2, D, BW), jnp.float32),
            pltpu.VMEM((SROWS, 128), jnp.float32),
            pltpu.VMEM((1, SROWS), jnp.int32),
            pltpu.SemaphoreType.DMA,
            pltpu.SemaphoreType.DMA,
            pltpu.SemaphoreType.DMA,
        ],
    )(_gather_body)
    g = gather(heads, tails, entity_t)

    norm = functools.partial(
        pl.kernel, mesh=mesh,
        out_type=jax.ShapeDtypeStruct((B,), jnp.float32),
        compiler_params=params,
        scratch_types=[
            pltpu.VMEM((BPW,), jnp.int32),
            pltpu.VMEM((D, NR), jnp.float32),
            pltpu.VMEM((2, 64, 128), jnp.float32),
            pltpu.VMEM((2, 64, 128), jnp.float32),
            pltpu.VMEM((BPW,), jnp.float32),
            pltpu.SemaphoreType.DMA,
            pltpu.SemaphoreType.DMA,
        ],
    )(_norm_body)
    return norm(relations, relation_t, g)


def kernel(heads, relations, tails, entity_emb, relation_emb):
    heads = heads.astype(jnp.int32)
    relations = relations.astype(jnp.int32)
    tails = tails.astype(jnp.int32)
    return _transe_sc(heads, relations, tails, entity_emb.T, relation_emb.T)


# speedup vs baseline: 1.0746x; 1.0142x over previous
import functools

import jax
import jax.numpy as jnp
from jax import lax
from jax.experimental import pallas as pl
from jax.experimental.pallas import tpu as pltpu
from jax.experimental.pallas import tpu_sc as plsc

B = 16384
D = 64
NE = 1000000
NR = 1000
NC = 2
NS = 16
NW = NC * NS
BPW = B // NW
L = 16

BW = 256
NFB = NE // BW
TAIL0 = NFB * BW
NBLK = NFB + 1
CAP = 32
MAXBPW = NFB // NW + 1
G1 = 2 * B
SPARE = G1
N1 = G1 + 128


SROWS = 128


def _flush(staging, posrow, out1_hbm, sem, iota):
    cp = pltpu.async_copy(staging, out1_hbm.at[posrow.at[0]], sem)
    cp.wait()
    for v in range(SROWS // L):
        posrow[0, pl.ds(v * L, L)] = jnp.full((L,), SPARE + v * L, jnp.int32) + iota


def _gather_body(heads_hbm, tails_hbm, et_hbm, out1_hbm,
                 src_v, ovf, buckets, bcnt, blockbuf, staging, posrow,
                 sem_a, sem_b, sem_s):
    wid = lax.axis_index("s") * NC + lax.axis_index("c")
    start = wid * 122 + jnp.minimum(wid, 2)
    cnt = 122 + (wid < 2).astype(jnp.int32)
    lo = start * BW
    hi = jnp.where(wid == NW - 1, NE, (start + cnt) * BW)
    iota = lax.iota(jnp.int32, L)

    for v in range(SROWS // L):
        posrow[0, pl.ds(v * L, L)] = jnp.full((L,), SPARE + v * L, jnp.int32) + iota
    for v in range(128 // L):
        bcnt[pl.ds(v * L, L)] = jnp.zeros((L,), jnp.int32)

    def filter_list(src_hbm, posbase, novf):
        pltpu.sync_copy(src_hbm, src_v)

        def fv(v, novf):
            idx = src_v[pl.ds(v * L, L)]
            inr = (idx >= lo) & (idx < hi)
            lidx = idx - lo
            pos = posbase + v * L + iota
            packed = (lidx << 16) | pos
            bkt = lidx >> 8
            base = plsc.load_gather(bcnt, [jnp.where(inr, bkt, 0)])
            rank, _ = plsc.scan_count(bkt, inr)
            slot = base + rank - 1
            keep = inr & (slot < CAP)
            spill = inr & (slot >= CAP)
            plsc.store_scatter(buckets, [bkt * CAP + slot], packed, mask=keep)
            plsc.addupdate_scatter(bcnt, [bkt], inr.astype(jnp.int32),
                                   mask=inr)
            pref = plsc.cumsum(spill.astype(jnp.int32))
            plsc.store_scatter(ovf, [novf + pref - 1], packed, mask=spill)
            return novf + pref[15]

        return lax.fori_loop(0, B // L, fv, novf)

    pltpu.async_copy(et_hbm.at[:, pl.ds(pl.multiple_of(start * BW, 128), BW)],
                     blockbuf.at[0], sem_a)

    novf = filter_list(heads_hbm, 0, jnp.int32(0))
    novf = filter_list(tails_hbm, B, novf)

    def extract_event(buf, hits16, mask, n):
        @pl.when(n > SROWS - L)
        def _():
            _flush(staging, posrow, out1_hbm, sem_s, iota)

        n = jnp.where(n > SROWS - L, 0, n)
        bidx = (hits16 >> 16) & (BW - 1)
        pos = hits16 & 0xFFFF
        pref = plsc.cumsum(mask.astype(jnp.int32))
        slot = n + pref - 1
        plsc.store_scatter(posrow, [jnp.zeros((L,), jnp.int32), slot], pos,
                           mask=mask)
        sbase = slot * 128
        dv = jnp.zeros((L,), jnp.int32)
        for d in range(D):
            vals = plsc.load_gather(buf, [dv, bidx])
            plsc.store_scatter(staging, [slot, dv], vals, mask=mask)
            dv = dv + 1
        return n + pref[15]

    def process_block(i, buf, n):
        cnt_i = plsc.load_gather(bcnt, [jnp.full((L,), i, jnp.int32)])[0]
        cnt_eff = jnp.minimum(cnt_i, CAP)

        def bg(g, n):
            hits16 = buckets[pl.ds(i * CAP + g * L, L)]
            mask = (g * L + iota) < cnt_eff
            return extract_event(buf, hits16, mask, n)

        n = lax.fori_loop(0, (cnt_eff + L - 1) // L, bg, n)

        def ov(v, n):
            hits16 = ovf[pl.ds(v * L, L)]
            mask = ((hits16 >> 16) >> 8) == i
            mask = mask & ((v * L + iota) < novf)
            return extract_event(buf, hits16, mask, n)

        n = lax.cond(cnt_i > CAP,
                     lambda n: lax.fori_loop(0, (novf + L - 1) // L, ov, n),
                     lambda n: n, n)
        return n

    def issue(i, buf, sem):
        blk = pl.multiple_of((start + i) * BW, 128)
        return pltpu.async_copy(et_hbm.at[:, pl.ds(blk, BW)], buf, sem)

    def wait_like(buf, sem):
        pltpu.make_async_copy(et_hbm.at[:, pl.ds(0, BW)], buf, sem).wait()

    def step(i2, n):
        i = i2 * 2

        def one(i, buf, sem, obuf, osem, n):
            @pl.when(i + 1 < cnt)
            def _():
                issue(i + 1, obuf, osem)
            wait_like(buf, sem)
            return process_block(i, buf, n)

        n = one(i, blockbuf.at[0], sem_a, blockbuf.at[1], sem_b, n)
        n = lax.cond(i + 1 < cnt,
                     lambda n: one(i + 1, blockbuf.at[1], sem_b,
                                   blockbuf.at[0], sem_a, n),
                     lambda n: n, n)
        return n

    n = lax.fori_loop(0, (cnt + 1) // 2, step, jnp.int32(0))

    @pl.when(wid == NW - 1)
    def _():
        toff = pl.multiple_of((hi >> 7) << 7, 128)
        pltpu.sync_copy(et_hbm.at[:, pl.ds(toff, 128)],
                        blockbuf.at[0, :, pl.ds(0, 128)])

    n = lax.cond(
        wid == NW - 1,
        lambda n: process_block(cnt, blockbuf.at[0], n),
        lambda n: n, n)

    @pl.when(n > 0)
    def _():
        _flush(staging, posrow, out1_hbm, sem_s, iota)


def _norm_body(rels_hbm, remb_hbm, g_hbm, out_hbm,
               idx_r, relbuf, hchunk, tchunk, sums, sem_h, sem_t):
    wid = lax.axis_index("s") * NC + lax.axis_index("c")
    base = wid * BPW
    iota = lax.iota(jnp.int32, L)

    CH = 64

    def issue(c, k):
        pltpu.async_copy(g_hbm.at[pl.ds(base + c * CH, CH), :],
                         hchunk.at[k], sem_h)
        pltpu.async_copy(g_hbm.at[pl.ds(B + base + c * CH, CH), :],
                         tchunk.at[k], sem_t)

    issue(0, 0)
    pltpu.sync_copy(rels_hbm.at[pl.ds(base, BPW)], idx_r)
    pltpu.sync_copy(remb_hbm, relbuf)

    def wait_chunk(k):
        pltpu.make_async_copy(g_hbm.at[pl.ds(0, CH), :], hchunk.at[k],
                              sem_h).wait()
        pltpu.make_async_copy(g_hbm.at[pl.ds(0, CH), :], tchunk.at[k],
                              sem_t).wait()

    def chunk(c, k, carry):
        @pl.when(c + 1 < BPW // CH)
        def _():
            issue(c + 1, 1 - k)
        wait_chunk(k)

        def grp(g, carry):
            row = g * L + iota
            rv = idx_r[pl.ds(c * CH + g * L, L)]
            accs = [jnp.zeros((L,), jnp.float32) for _ in range(4)]
            dv = jnp.zeros((L,), jnp.int32)
            for d in range(D):
                h = plsc.load_gather(hchunk.at[k], [row, dv])
                t = plsc.load_gather(tchunk.at[k], [row, dv])
                r = plsc.load_gather(relbuf, [dv, rv])
                dd = (h + r) - t
                accs[d % 4] = accs[d % 4] + dd * dd
                dv = dv + 1
            x = (accs[0] + accs[1]) + (accs[2] + accs[3])
            xi = plsc.bitcast(x, jnp.int32)
            y = plsc.bitcast(jnp.int32(0x5F3759DF) - (xi >> 1), jnp.float32)
            for _ in range(3):
                y = y * (1.5 - 0.5 * x * y * y)
            sums[pl.ds(c * CH + g * L, L)] = x * y
            return carry

        return lax.fori_loop(0, CH // L, grp, carry)

    def pair(c2, carry):
        carry = chunk(c2 * 2, 0, carry)
        return chunk(c2 * 2 + 1, 1, carry)

    lax.fori_loop(0, BPW // CH // 2, pair, 0)
    pltpu.sync_copy(sums, out_hbm.at[pl.ds(base, BPW)])


@jax.jit
def _transe_sc(heads, relations, tails, entity_t, relation_t):
    mesh = plsc.VectorSubcoreMesh(core_axis_name="c", subcore_axis_name="s")
    params = pltpu.CompilerParams(
        needs_layout_passes=False, use_tc_tiling_on_sc=True)

    gather = functools.partial(
        pl.kernel, mesh=mesh,
        out_type=jax.ShapeDtypeStruct((N1, 128), jnp.float32),
        compiler_params=params,
        scratch_types=[
            pltpu.VMEM((B,), jnp.int32),
            pltpu.VMEM((G1,), jnp.int32),
            pltpu.VMEM((MAXBPW * CAP,), jnp.int32),
            pltpu.VMEM((128,), jnp.int32),
            pltpu.VMEM((2, D, BW), jnp.float32),
            pltpu.VMEM((SROWS, 128), jnp.float32),
            pltpu.VMEM((1, SROWS), jnp.int32),
            pltpu.SemaphoreType.DMA,
            pltpu.SemaphoreType.DMA,
            pltpu.SemaphoreType.DMA,
        ],
    )(_gather_body)
    g = gather(heads, tails, entity_t)

    norm = functools.partial(
        pl.kernel, mesh=mesh,
        out_type=jax.ShapeDtypeStruct((B,), jnp.float32),
        compiler_params=params,
        scratch_types=[
            pltpu.VMEM((BPW,), jnp.int32),
            pltpu.VMEM((D, NR), jnp.float32),
            pltpu.VMEM((2, 64, 128), jnp.float32),
            pltpu.VMEM((2, 64, 128), jnp.float32),
            pltpu.VMEM((BPW,), jnp.float32),
            pltpu.SemaphoreType.DMA,
            pltpu.SemaphoreType.DMA,
        ],
    )(_norm_body)
    return norm(relations, relation_t, g)


def kernel(heads, relations, tails, entity_emb, relation_emb):
    heads = heads.astype(jnp.int32)
    relations = relations.astype(jnp.int32)
    tails = tails.astype(jnp.int32)
    return _transe_sc(heads, relations, tails, entity_emb.T, relation_emb.T)
